# baseline (speedup 1.0000x reference)
import functools
import math

import jax
import jax.numpy as jnp
from jax import lax
from jax.experimental import pallas as pl
from jax.experimental.pallas import tpu as pltpu
from jax.experimental.pallas import tpu_sc as plsc

_HALF_LOG_2PI = 0.5 * math.log(2.0 * math.pi)
_LN2 = math.log(2.0)


def _vlog(v):
    bits = plsc.bitcast(v, jnp.int32)
    e = (bits >> 23) - 127
    m = plsc.bitcast((bits & 0x007FFFFF) | 0x3F800000, jnp.float32)
    r = (m - 1.0) / (m + 1.0)
    r2 = r * r
    p = r2 * (1.0 / 7.0) + (1.0 / 5.0)
    p = r2 * p + (1.0 / 3.0)
    p = r2 * p + 1.0
    return e.astype(jnp.float32) * _LN2 + 2.0 * r * p


def sc_mog(z, means, logvars, w, B, L, K):
    info = plsc.get_sparse_core_info()
    NC, NS, LN = info.num_cores, info.num_subcores, info.num_lanes
    NW = NC * NS
    RW = B // NW
    NT = RW // 16
    NLG = L // LN
    mesh = plsc.VectorSubcoreMesh(core_axis_name="c", subcore_axis_name="s")

    @functools.partial(
        pl.kernel,
        mesh=mesh,
        compiler_params=pltpu.CompilerParams(needs_layout_passes=False),
        out_type=jax.ShapeDtypeStruct((L * B,), jnp.float32),
        scratch_types=[
            pltpu.VMEM((RW * L,), jnp.float32),
            pltpu.VMEM((L * RW,), jnp.float32),
            pltpu.VMEM((K * L,), jnp.float32),
            pltpu.VMEM((K * L,), jnp.float32),
            pltpu.VMEM((K * L,), jnp.float32),
            pltpu.VMEM((K * L,), jnp.float32),
            pltpu.VMEM((K * L,), jnp.float32),
            pltpu.VMEM((K,), jnp.float32),
            pltpu.VMEM((16 * 16,), jnp.float32),
        ],
    )
    def _sc(z_hbm, mu_hbm, lv_hbm, w_hbm, out_hbm,
            z_v, yt_v, a_v, b_v, c_v, mu_v, lv_v, w_v, t16_v):
        wid = lax.axis_index("s") * NC + lax.axis_index("c")
        base = wid * RW

        pltpu.sync_copy(z_hbm.at[pl.ds(base * L, RW * L)], z_v)
        pltpu.sync_copy(mu_hbm, mu_v)
        pltpu.sync_copy(lv_hbm, lv_v)
        pltpu.sync_copy(w_hbm, w_v)

        iota = lax.iota(jnp.int32, LN)

        def splat_reduce(v, op):
            for s in (8, 4, 2, 1):
                t16_v[pl.ds(0, LN)] = v
                v = op(v, plsc.load_gather(t16_v, [iota ^ s]))
            return v

        wc = [w_v[pl.ds(LN * i, LN)] for i in range(K // LN)]
        wt = wc[0]
        for i in range(1, K // LN):
            wt = jnp.maximum(wt, wc[i])
        wm = splat_reduce(wt, jnp.maximum)
        sv = jnp.exp(wc[0] - wm)
        for i in range(1, K // LN):
            sv = sv + jnp.exp(wc[i] - wm)
        lse_v = _vlog(splat_reduce(sv, jnp.add))

        neg_inf = jnp.full((LN,), -1e30, jnp.float32)

        def coef_body(k, mcar):
            wk = plsc.load_gather(w_v, [jnp.full((LN,), k, jnp.int32)]) - wm
            out = []
            for g in range(NLG):
                off = k * L + g * LN
                muc = mu_v[pl.ds(off, LN)]
                lvc = lv_v[pl.ds(off, LN)]
                a = -0.5 * jnp.exp(-lvc)
                b = (-2.0 * a) * muc
                c = (wk - _HALF_LOG_2PI - 0.5 * lvc) + a * muc * muc
                a_v[pl.ds(off, LN)] = a
                b_v[pl.ds(off, LN)] = b
                c_v[pl.ds(off, LN)] = c
                out.append(jnp.maximum(mcar[g], c))
            return tuple(out)

        M = lax.fori_loop(0, K, coef_body, tuple(neg_inf for _ in range(NLG)))

        def shift_body(k, _):
            for g in range(NLG):
                off = k * L + g * LN
                c_v[pl.ds(off, LN)] = c_v[pl.ds(off, LN)] - M[g]
            return 0

        lax.fori_loop(0, K, shift_body, 0)

        colbase = iota * LN

        def tile_body(t, _):
            b0 = t * 16
            for g in range(NLG):
                zs = tuple(z_v[pl.ds((b0 + r) * L + g * LN, LN)]
                           for r in range(16))
                z2s = tuple(zz * zz for zz in zs)

                def k_body(k, ss):
                    off = k * L + g * LN
                    a = a_v[pl.ds(off, LN)]
                    b = b_v[pl.ds(off, LN)]
                    c = c_v[pl.ds(off, LN)]
                    return tuple(ss[r] + jnp.exp((a * z2s[r] + b * zs[r]) + c)
                                 for r in range(16))

                zero = jnp.zeros((LN,), jnp.float32)
                ss = lax.fori_loop(0, K, k_body, tuple(zero for _ in range(16)))

                for r in range(16):
                    y = (lse_v - M[g]) - _vlog(ss[r])
                    t16_v[pl.ds(r * 16, 16)] = y
                for cc in range(16):
                    col = plsc.load_gather(t16_v, [colbase + cc])
                    yt_v[pl.ds((g * LN + cc) * RW + b0, 16)] = col
            return 0

        lax.fori_loop(0, NT, tile_body, 0)

        def out_body(l, _):
            pltpu.sync_copy(yt_v.at[pl.ds(l * RW, RW)],
                            out_hbm.at[pl.ds(l * B + base, RW)])
            return 0

        lax.fori_loop(0, L, out_body, 0)

    return _sc(z.reshape(B * L), means.reshape(K * L),
               logvars.reshape(K * L), w.reshape(K))


_HALF_LOG_2PI = 0.5 * math.log(2.0 * math.pi)
_LOG2E = 1.4426950408889634
_LN2 = 0.6931471805599453
_CH = 128


def _mog_block(za_ref, zb_ref, means_ref, logvars_ref, w_ref, out_ref):
    mu0 = means_ref[...]
    lv0 = logvars_ref[...]
    w = w_ref[...]
    K, L = mu0.shape
    R = za_ref.shape[0]

    wmax = jnp.max(w, axis=0, keepdims=True)
    logw = (w - wmax) - jnp.log(jnp.sum(jnp.exp(w - wmax), axis=0, keepdims=True))

    mu = jnp.concatenate([mu0, mu0], axis=1)
    lv = jnp.concatenate([lv0, lv0], axis=1)
    lw = jnp.concatenate([logw, logw], axis=1)

    A = -0.5 * jnp.exp(-lv)
    Bc = (-2.0 * A) * mu
    C = (lw - _HALF_LOG_2PI - 0.5 * lv) + A * mu * mu
    M = jnp.max(C, axis=0, keepdims=True)
    A2 = (A * _LOG2E).astype(jnp.bfloat16)
    B2 = (Bc * _LOG2E).astype(jnp.bfloat16)
    C2 = ((C - M) * _LOG2E).astype(jnp.bfloat16)

    smin = jnp.float32(jnp.inf)
    for c in range(R // _CH):
        rows = pl.ds(c * _CH, _CH)
        z = jnp.concatenate([za_ref[rows, :], zb_ref[rows, :]], axis=1).astype(jnp.bfloat16)
        z2 = z * z
        s = jnp.zeros(z.shape, jnp.bfloat16)
        for k in range(K):
            x = A2[k : k + 1, :] * z2 + B2[k : k + 1, :] * z + C2[k : k + 1, :]
            s = s + jnp.exp2(x)
        sf = s.astype(jnp.float32)
        smin = jnp.minimum(smin, jnp.min(sf))
        yt = (-(M + _LN2 * jnp.log2(sf))).T
        out_ref[:, pl.ds(c * _CH, _CH)] = yt[:L, :]
        out_ref[:, pl.ds(R + c * _CH, _CH)] = yt[L:, :]

    @pl.when(smin < 1e-30)
    def _fixup():
        for c in range(R // _CH):
            rows = pl.ds(c * _CH, _CH)
            z = jnp.concatenate([za_ref[rows, :], zb_ref[rows, :]], axis=1)
            z2 = z * z
            Af = A2.astype(jnp.float32)
            Bf = B2.astype(jnp.float32)
            Cf = C2.astype(jnp.float32)
            m = Af[0:1, :] * z2 + Bf[0:1, :] * z + Cf[0:1, :]
            for k in range(1, K):
                x = Af[k : k + 1, :] * z2 + Bf[k : k + 1, :] * z + Cf[k : k + 1, :]
                m = jnp.maximum(m, x)
            s2 = jnp.zeros(z.shape, jnp.float32)
            for k in range(K):
                x = Af[k : k + 1, :] * z2 + Bf[k : k + 1, :] * z + Cf[k : k + 1, :]
                s2 = s2 + jnp.exp2(x - m)
            yt2 = (-(M + _LN2 * (m + jnp.log2(s2)))).T
            out_ref[:, pl.ds(c * _CH, _CH)] = yt2[:L, :]
            out_ref[:, pl.ds(R + c * _CH, _CH)] = yt2[L:, :]


def _tc_kernel(z, means, logvars, w):
    B, L = z.shape
    K = means.shape[0]
    R = 1024
    nblk = B // (2 * R)
    w_b = jnp.broadcast_to(w.reshape(K, 1), (K, L))
    return pl.pallas_call(
        _mog_block,
        grid=(nblk,),
        in_specs=[
            pl.BlockSpec((R, L), lambda i: (2 * i, 0)),
            pl.BlockSpec((R, L), lambda i: (2 * i + 1, 0)),
            pl.BlockSpec((K, L), lambda i: (0, 0)),
            pl.BlockSpec((K, L), lambda i: (0, 0)),
            pl.BlockSpec((K, L), lambda i: (0, 0)),
        ],
        out_specs=pl.BlockSpec((L, 2 * R), lambda i: (0, i)),
        out_shape=jax.ShapeDtypeStruct((L, B), jnp.float32),
        compiler_params=pltpu.CompilerParams(
            dimension_semantics=("arbitrary",),
        ),
    )(z, z, means, logvars, w_b)


_SC_ROWS = 2048


def kernel(z, means, logvars, w):
    B, L = z.shape
    K = means.shape[0]
    sc_flat = sc_mog(z[:_SC_ROWS], means, logvars, w, _SC_ROWS, L, K)
    sc_out = sc_flat.reshape(L, _SC_ROWS)
    tc_out = _tc_kernel(z[_SC_ROWS:], means, logvars, w)
    return jnp.concatenate([sc_out, tc_out], axis=1)


# speedup vs baseline: 1.5187x; 1.5187x over previous
import math

import jax
import jax.numpy as jnp
from jax.experimental import pallas as pl
from jax.experimental.pallas import tpu as pltpu

_HALF_LOG_2PI = 0.5 * math.log(2.0 * math.pi)
_LOG2E = 1.4426950408889634
_LN2 = 0.6931471805599453
_CH = 128


def _mog_block(za_ref, zb_ref, means_ref, logvars_ref, w_ref, out_ref):
    mu0 = means_ref[...]
    lv0 = logvars_ref[...]
    w = w_ref[...]
    K, L = mu0.shape
    R = za_ref.shape[0]

    wmax = jnp.max(w, axis=0, keepdims=True)
    logw = (w - wmax) - jnp.log(jnp.sum(jnp.exp(w - wmax), axis=0, keepdims=True))

    mu = jnp.concatenate([mu0, mu0], axis=1)
    lv = jnp.concatenate([lv0, lv0], axis=1)
    lw = jnp.concatenate([logw, logw], axis=1)

    A = -0.5 * jnp.exp(-lv)
    Bc = (-2.0 * A) * mu
    C = (lw - _HALF_LOG_2PI - 0.5 * lv) + A * mu * mu
    M = jnp.max(C, axis=0, keepdims=True)
    A2 = (A * _LOG2E).astype(jnp.bfloat16)
    B2 = (Bc * _LOG2E).astype(jnp.bfloat16)
    C2 = ((C - M) * _LOG2E).astype(jnp.bfloat16)

    smin = jnp.float32(jnp.inf)
    for c in range(R // _CH):
        rows = pl.ds(c * _CH, _CH)
        z = jnp.concatenate([za_ref[rows, :], zb_ref[rows, :]], axis=1).astype(jnp.bfloat16)
        z2 = z * z
        s = jnp.zeros(z.shape, jnp.bfloat16)
        for k in range(K):
            x = A2[k : k + 1, :] * z2 + B2[k : k + 1, :] * z + C2[k : k + 1, :]
            s = s + jnp.exp2(x)
        sf = s.astype(jnp.float32)
        smin = jnp.minimum(smin, jnp.min(sf))
        yt = (-(M + _LN2 * jnp.log2(sf))).T
        out_ref[:, pl.ds(c * _CH, _CH)] = yt[:L, :]
        out_ref[:, pl.ds(R + c * _CH, _CH)] = yt[L:, :]

    @pl.when(smin < 1e-30)
    def _fixup():
        for c in range(R // _CH):
            rows = pl.ds(c * _CH, _CH)
            z = jnp.concatenate([za_ref[rows, :], zb_ref[rows, :]], axis=1)
            z2 = z * z
            Af = A2.astype(jnp.float32)
            Bf = B2.astype(jnp.float32)
            Cf = C2.astype(jnp.float32)
            m = Af[0:1, :] * z2 + Bf[0:1, :] * z + Cf[0:1, :]
            for k in range(1, K):
                x = Af[k : k + 1, :] * z2 + Bf[k : k + 1, :] * z + Cf[k : k + 1, :]
                m = jnp.maximum(m, x)
            s2 = jnp.zeros(z.shape, jnp.float32)
            for k in range(K):
                x = Af[k : k + 1, :] * z2 + Bf[k : k + 1, :] * z + Cf[k : k + 1, :]
                s2 = s2 + jnp.exp2(x - m)
            yt2 = (-(M + _LN2 * (m + jnp.log2(s2)))).T
            out_ref[:, pl.ds(c * _CH, _CH)] = yt2[:L, :]
            out_ref[:, pl.ds(R + c * _CH, _CH)] = yt2[L:, :]


def kernel(z, means, logvars, w):
    B, L = z.shape
    K = means.shape[0]
    R = 2048
    nblk = B // (2 * R)
    w_b = jnp.broadcast_to(w.reshape(K, 1), (K, L))
    return pl.pallas_call(
        _mog_block,
        grid=(nblk,),
        in_specs=[
            pl.BlockSpec((R, L), lambda i: (2 * i, 0)),
            pl.BlockSpec((R, L), lambda i: (2 * i + 1, 0)),
            pl.BlockSpec((K, L), lambda i: (0, 0)),
            pl.BlockSpec((K, L), lambda i: (0, 0)),
            pl.BlockSpec((K, L), lambda i: (0, 0)),
        ],
        out_specs=pl.BlockSpec((L, 2 * R), lambda i: (0, i)),
        out_shape=jax.ShapeDtypeStruct((L, B), jnp.float32),
        compiler_params=pltpu.CompilerParams(
            dimension_semantics=("arbitrary",),
        ),
    )(z, z, means, logvars, w_b)
